# initial kernel scaffold (unmeasured)
import jax
import jax.numpy as jnp
from jax import lax
from jax.experimental import pallas as pl
from jax.experimental.pallas import tpu as pltpu


def kernel(
    x,
):
    def body(*refs):
        pass

    out_shape = jax.ShapeDtypeStruct(..., jnp.float32)
    return pl.pallas_call(body, out_shape=out_shape)(...)


# baseline (device time: 424449 ns/iter reference)
import jax
import jax.numpy as jnp
from jax import lax
from jax.experimental import pallas as pl
from jax.experimental.pallas import tpu as pltpu

M = 4096
N_HALF = 2048
C = 8
R = M // C
NLOAD = 4


def kernel(x):
    def body(x_hbm, out_hbm, xin, xsend, xrecv,
             load_sems, xsend_sems, xrecv_sems,
             ysend_sems, yrecv_sems, outcpy_sems):
        my_x = lax.axis_index("x")
        my_y = lax.axis_index("y")

        barrier_sem = pltpu.get_barrier_semaphore()
        pl.semaphore_signal(
            barrier_sem, inc=1,
            device_id=(1 - my_x, my_y), device_id_type=pl.DeviceIdType.MESH,
        )
        pl.semaphore_signal(
            barrier_sem, inc=1,
            device_id=(my_x, 1 - my_y), device_id_type=pl.DeviceIdType.MESH,
        )
        pl.semaphore_wait(barrier_sem, 2)

        col = pl.ds(my_y * N_HALF, N_HALF)

        def start_load(c):
            cp = pltpu.make_async_copy(
                x_hbm.at[0, pl.ds(c * R, R), :],
                xin.at[c % NLOAD],
                load_sems.at[c % NLOAD],
            )
            cp.start()
            return cp

        for c in range(C):
            rows = pl.ds(c * R, R)
            start_load(c).wait()
            xsend[rows] = xin[c % NLOAD].astype(jnp.bfloat16)

            rdma_x = pltpu.make_async_remote_copy(
                src_ref=xsend.at[rows],
                dst_ref=xrecv.at[rows],
                send_sem=xsend_sems.at[c],
                recv_sem=xrecv_sems.at[c],
                device_id=(1 - my_x, my_y),
                device_id_type=pl.DeviceIdType.MESH,
            )
            rdma_x.start()
            rdma_x.wait()

            xsend[rows] = xsend[rows] + xrecv[rows]

            outcpy = pltpu.make_async_copy(
                xsend.at[rows], out_hbm.at[rows, col], outcpy_sems.at[c]
            )
            outcpy.start()

            rdma_y = pltpu.make_async_remote_copy(
                src_ref=xsend.at[rows],
                dst_ref=out_hbm.at[rows, col],
                send_sem=ysend_sems.at[c],
                recv_sem=yrecv_sems.at[c],
                device_id=(my_x, 1 - my_y),
                device_id_type=pl.DeviceIdType.MESH,
            )
            rdma_y.start()
            rdma_y.wait_send()
            outcpy.wait()

        for c in range(C):
            pltpu.make_async_remote_copy(
                src_ref=xsend.at[pl.ds(c * R, R)],
                dst_ref=out_hbm.at[pl.ds(c * R, R), col],
                send_sem=ysend_sems.at[c],
                recv_sem=yrecv_sems.at[c],
                device_id=(my_x, 1 - my_y),
                device_id_type=pl.DeviceIdType.MESH,
            ).wait_recv()

    return pl.pallas_call(
        body,
        out_shape=jax.ShapeDtypeStruct((M, 2 * N_HALF), jnp.bfloat16),
        in_specs=[pl.BlockSpec(memory_space=pl.ANY)],
        out_specs=pl.BlockSpec(memory_space=pl.ANY),
        scratch_shapes=[
            pltpu.VMEM((NLOAD, R, N_HALF), jnp.float32),
            pltpu.VMEM((M, N_HALF), jnp.bfloat16),
            pltpu.VMEM((M, N_HALF), jnp.bfloat16),
            pltpu.SemaphoreType.DMA((NLOAD,)),
            pltpu.SemaphoreType.DMA((C,)),
            pltpu.SemaphoreType.DMA((C,)),
            pltpu.SemaphoreType.DMA((C,)),
            pltpu.SemaphoreType.DMA((C,)),
            pltpu.SemaphoreType.DMA((C,)),
        ],
        compiler_params=pltpu.CompilerParams(
            collective_id=0, vmem_limit_bytes=56 * 1024 * 1024
        ),
    )(x)


# device time: 221131 ns/iter; 1.9194x vs baseline; 1.9194x over previous
import jax
import jax.numpy as jnp
from jax import lax
from jax.experimental import pallas as pl
from jax.experimental.pallas import tpu as pltpu

M = 4096
N_HALF = 2048
C = 16
R = M // C
NLOAD = 4
LOOKAHEAD = NLOAD - 1


def kernel(x):
    def body(x_hbm, out_hbm, xin, xsend, xrecv,
             load_sems, xsend_sems, xrecv_sems,
             ysend_sems, yrecv_sems, outcpy_sems):
        my_x = lax.axis_index("x")
        my_y = lax.axis_index("y")

        barrier_sem = pltpu.get_barrier_semaphore()
        pl.semaphore_signal(
            barrier_sem, inc=1,
            device_id=(1 - my_x, my_y), device_id_type=pl.DeviceIdType.MESH,
        )
        pl.semaphore_signal(
            barrier_sem, inc=1,
            device_id=(my_x, 1 - my_y), device_id_type=pl.DeviceIdType.MESH,
        )
        pl.semaphore_wait(barrier_sem, 2)

        col = pl.ds(my_y * N_HALF, N_HALF)

        def start_load(c):
            cp = pltpu.make_async_copy(
                x_hbm.at[0, pl.ds(c * R, R), :],
                xin.at[c % NLOAD],
                load_sems.at[c % NLOAD],
            )
            cp.start()
            return cp

        def make_rdma_x(c):
            rows = pl.ds(c * R, R)
            return pltpu.make_async_remote_copy(
                src_ref=xsend.at[rows],
                dst_ref=xrecv.at[rows],
                send_sem=xsend_sems.at[c],
                recv_sem=xrecv_sems.at[c],
                device_id=(1 - my_x, my_y),
                device_id_type=pl.DeviceIdType.MESH,
            )

        def make_rdma_y(c):
            rows = pl.ds(c * R, R)
            return pltpu.make_async_remote_copy(
                src_ref=xsend.at[rows],
                dst_ref=out_hbm.at[rows, col],
                send_sem=ysend_sems.at[c],
                recv_sem=yrecv_sems.at[c],
                device_id=(my_x, 1 - my_y),
                device_id_type=pl.DeviceIdType.MESH,
            )

        def finish(c):
            rows = pl.ds(c * R, R)
            rdma_x = make_rdma_x(c)
            rdma_x.wait_send()
            rdma_x.wait_recv()
            xsend[rows] = xsend[rows] + xrecv[rows]
            pltpu.make_async_copy(
                xsend.at[rows], out_hbm.at[rows, col], outcpy_sems.at[c]
            ).start()
            make_rdma_y(c).start()

        for c in range(min(LOOKAHEAD, C)):
            start_load(c)
        for c in range(C):
            if c + LOOKAHEAD < C:
                start_load(c + LOOKAHEAD)
            pltpu.make_async_copy(
                x_hbm.at[0, pl.ds(c * R, R), :],
                xin.at[c % NLOAD],
                load_sems.at[c % NLOAD],
            ).wait()
            xsend[pl.ds(c * R, R)] = xin[c % NLOAD].astype(jnp.bfloat16)
            make_rdma_x(c).start()
            if c >= 1:
                finish(c - 1)
        finish(C - 1)

        for c in range(C):
            rows = pl.ds(c * R, R)
            rdma_y = make_rdma_y(c)
            rdma_y.wait_send()
            rdma_y.wait_recv()
            pltpu.make_async_copy(
                xsend.at[rows], out_hbm.at[rows, col], outcpy_sems.at[c]
            ).wait()

    return pl.pallas_call(
        body,
        out_shape=jax.ShapeDtypeStruct((M, 2 * N_HALF), jnp.bfloat16),
        in_specs=[pl.BlockSpec(memory_space=pl.ANY)],
        out_specs=pl.BlockSpec(memory_space=pl.ANY),
        scratch_shapes=[
            pltpu.VMEM((NLOAD, R, N_HALF), jnp.float32),
            pltpu.VMEM((M, N_HALF), jnp.bfloat16),
            pltpu.VMEM((M, N_HALF), jnp.bfloat16),
            pltpu.SemaphoreType.DMA((NLOAD,)),
            pltpu.SemaphoreType.DMA((C,)),
            pltpu.SemaphoreType.DMA((C,)),
            pltpu.SemaphoreType.DMA((C,)),
            pltpu.SemaphoreType.DMA((C,)),
            pltpu.SemaphoreType.DMA((C,)),
        ],
        compiler_params=pltpu.CompilerParams(
            collective_id=0, vmem_limit_bytes=56 * 1024 * 1024
        ),
    )(x)


# device time: 215700 ns/iter; 1.9678x vs baseline; 1.0252x over previous
import jax
import jax.numpy as jnp
from jax import lax
from jax.experimental import pallas as pl
from jax.experimental.pallas import tpu as pltpu

M = 4096
N_HALF = 2048
C = 32
R = M // C
NLOAD = 4
LOOKAHEAD = NLOAD - 1


def kernel(x):
    def body(x_hbm, out_hbm, xin, xsend, xrecv,
             load_sems, xsend_sems, xrecv_sems,
             ysend_sems, yrecv_sems, outcpy_sems):
        my_x = lax.axis_index("x")
        my_y = lax.axis_index("y")

        barrier_sem = pltpu.get_barrier_semaphore()
        pl.semaphore_signal(
            barrier_sem, inc=1,
            device_id=(1 - my_x, my_y), device_id_type=pl.DeviceIdType.MESH,
        )
        pl.semaphore_signal(
            barrier_sem, inc=1,
            device_id=(my_x, 1 - my_y), device_id_type=pl.DeviceIdType.MESH,
        )
        pl.semaphore_wait(barrier_sem, 2)

        col = pl.ds(my_y * N_HALF, N_HALF)

        def start_load(c):
            cp = pltpu.make_async_copy(
                x_hbm.at[0, pl.ds(c * R, R), :],
                xin.at[c % NLOAD],
                load_sems.at[c % NLOAD],
            )
            cp.start()
            return cp

        def make_rdma_x(c):
            rows = pl.ds(c * R, R)
            return pltpu.make_async_remote_copy(
                src_ref=xsend.at[rows],
                dst_ref=xrecv.at[rows],
                send_sem=xsend_sems.at[c],
                recv_sem=xrecv_sems.at[c],
                device_id=(1 - my_x, my_y),
                device_id_type=pl.DeviceIdType.MESH,
            )

        def make_rdma_y(c):
            rows = pl.ds(c * R, R)
            return pltpu.make_async_remote_copy(
                src_ref=xsend.at[rows],
                dst_ref=out_hbm.at[rows, col],
                send_sem=ysend_sems.at[c],
                recv_sem=yrecv_sems.at[c],
                device_id=(my_x, 1 - my_y),
                device_id_type=pl.DeviceIdType.MESH,
            )

        def finish(c):
            rows = pl.ds(c * R, R)
            rdma_x = make_rdma_x(c)
            rdma_x.wait_send()
            rdma_x.wait_recv()
            xsend[rows] = xsend[rows] + xrecv[rows]
            pltpu.make_async_copy(
                xsend.at[rows], out_hbm.at[rows, col], outcpy_sems.at[c]
            ).start()
            make_rdma_y(c).start()

        for c in range(min(LOOKAHEAD, C)):
            start_load(c)
        for c in range(C):
            if c + LOOKAHEAD < C:
                start_load(c + LOOKAHEAD)
            pltpu.make_async_copy(
                x_hbm.at[0, pl.ds(c * R, R), :],
                xin.at[c % NLOAD],
                load_sems.at[c % NLOAD],
            ).wait()
            xsend[pl.ds(c * R, R)] = xin[c % NLOAD].astype(jnp.bfloat16)
            make_rdma_x(c).start()
            if c >= 1:
                finish(c - 1)
        finish(C - 1)

        for c in range(C):
            rows = pl.ds(c * R, R)
            rdma_y = make_rdma_y(c)
            rdma_y.wait_send()
            rdma_y.wait_recv()
            pltpu.make_async_copy(
                xsend.at[rows], out_hbm.at[rows, col], outcpy_sems.at[c]
            ).wait()

    return pl.pallas_call(
        body,
        out_shape=jax.ShapeDtypeStruct((M, 2 * N_HALF), jnp.bfloat16),
        in_specs=[pl.BlockSpec(memory_space=pl.ANY)],
        out_specs=pl.BlockSpec(memory_space=pl.ANY),
        scratch_shapes=[
            pltpu.VMEM((NLOAD, R, N_HALF), jnp.float32),
            pltpu.VMEM((M, N_HALF), jnp.bfloat16),
            pltpu.VMEM((M, N_HALF), jnp.bfloat16),
            pltpu.SemaphoreType.DMA((NLOAD,)),
            pltpu.SemaphoreType.DMA((C,)),
            pltpu.SemaphoreType.DMA((C,)),
            pltpu.SemaphoreType.DMA((C,)),
            pltpu.SemaphoreType.DMA((C,)),
            pltpu.SemaphoreType.DMA((C,)),
        ],
        compiler_params=pltpu.CompilerParams(
            collective_id=0, vmem_limit_bytes=56 * 1024 * 1024
        ),
    )(x)
